# baseline (speedup 1.0000x reference)
import functools

import jax
import jax.numpy as jnp
from jax import lax
from jax.experimental import pallas as pl
from jax.experimental.pallas import tpu as pltpu
from jax.experimental.pallas import tpu_sc as plsc

EMBED = 64
NB = 4096
NT = 200
B_TOTAL = NB * NT
NUM_WORKERS = 32
B_PER_W = B_TOTAL // NUM_WORKERS
CHUNK = 512
N_CHUNKS = B_PER_W // CHUNK
NBUF = 2


def _gather_body(events_hbm, table_hbm, out_hbm, idx_v, rows_v, gsem, wsem):
    wid = lax.axis_index("s") * 2 + lax.axis_index("c")

    pltpu.sync_copy(events_hbm.at[wid], idx_v)

    def start_gather(j, b):
        pltpu.make_async_copy(
            table_hbm.at[idx_v.at[pl.ds(j * CHUNK, CHUNK)]],
            rows_v.at[b], gsem.at[b]
        ).start()

    def wait_gather(b):
        pltpu.make_async_copy(
            table_hbm.at[idx_v.at[pl.ds(0, CHUNK)]], rows_v.at[b], gsem.at[b]
        ).wait()

    def start_write(j, b):
        pltpu.make_async_copy(
            rows_v.at[b], out_hbm.at[wid, j], wsem.at[b]
        ).start()

    def wait_write(b):
        pltpu.make_async_copy(
            rows_v.at[b], out_hbm.at[wid, 0], wsem.at[b]
        ).wait()

    for b in range(NBUF):
        start_gather(b, b)

    def group(gg, _):
        for b in range(NBUF):
            j = gg * NBUF + b
            wait_gather(b)
            start_write(j, b)
            wait_write(b)
            start_gather(j + NBUF, b)
        return ()

    n_full = N_CHUNKS // NBUF - 1
    lax.fori_loop(0, n_full, group, ())

    for b in range(NBUF):
        j = n_full * NBUF + b
        wait_gather(b)
        start_write(j, b)
        wait_write(b)


@jax.jit
def _run(events_flat, table):
    mesh = plsc.VectorSubcoreMesh(core_axis_name="c", subcore_axis_name="s")
    kern = functools.partial(
        pl.kernel,
        out_type=jax.ShapeDtypeStruct((NUM_WORKERS, N_CHUNKS, CHUNK, EMBED),
                                      jnp.float32),
        mesh=mesh,
        scratch_types=[
            pltpu.VMEM((B_PER_W,), jnp.int32),
            pltpu.VMEM((NBUF, CHUNK, EMBED), jnp.float32),
            pltpu.SemaphoreType.DMA((NBUF,)),
            pltpu.SemaphoreType.DMA((NBUF,)),
        ],
        compiler_params=pltpu.CompilerParams(use_tc_tiling_on_sc=False),
    )(_gather_body)
    return kern(events_flat, table)


def kernel(events, table):
    events_flat = events.reshape(NUM_WORKERS, B_PER_W)
    out = _run(events_flat, table)
    return out.reshape(NB, NT, EMBED)
